# Initial kernel scaffold; baseline (speedup 1.0000x reference)
import jax
import jax.numpy as jnp
from jax.experimental import pallas as pl


def kernel(op_ids, node_feats, edge_index, emb, pre_w1, pre_b1, pre_w2, pre_b2, gc_w1, gc_b1, gc_w2, gc_b2, post_w1, post_b1, post_w2, post_b2):
    raise NotImplementedError("write your pallas kernel here")


# speedup vs baseline: 3.8972x; 3.8972x over previous
import functools

import jax
import jax.numpy as jnp
from jax import lax
from jax.experimental import pallas as pl
from jax.experimental.pallas import tpu as pltpu
from jax.experimental.pallas import tpu_sc as plsc

_N = 50000
_E = 800000
_H = 64
_HE = 8
_NP = 50176
_NOP = 53248
_EP = 802816
_ECH = 392
_RPT = _NP // 16
_B = 3136
_GB = _NP // _B


def _mesh():
    return plsc.VectorSubcoreMesh(core_axis_name="c", subcore_axis_name="s")


def _leaky(t):
    return jnp.where(t >= 0, t, 0.2 * t)


def _sc_deg_emb(srcp, opi, emb):
    out_type = [
        jax.ShapeDtypeStruct((_NP, 16), jnp.float32),
        jax.ShapeDtypeStruct((_NP, 16), jnp.float32),
        jax.ShapeDtypeStruct((_NOP, 32), jnp.float32),
    ]

    @functools.partial(
        pl.kernel, mesh=_mesh(), out_type=out_type,
        compiler_params=pltpu.CompilerParams(use_tc_tiling_on_sc=False),
        scratch_types=[
            pltpu.VMEM((_ECH, 128), jnp.int32),
            pltpu.VMEM((13, 128), jnp.int32),
            pltpu.VMEM((128, 16), jnp.float32),
            pltpu.VMEM((392, 16), jnp.float32),
            pltpu.VMEM((128, 32), jnp.float32),
            pltpu.VMEM_SHARED((_NP, 16), jnp.float32),
            pltpu.SemaphoreType.DMA,
        ])
    def k(srch, opih, embh, d0o, d1o, opeo, sslab, oslab, ones, halfb,
          obuf, accd, sem):
        c = lax.axis_index("c")
        s = lax.axis_index("s")
        w = 2 * s + c
        pltpu.sync_copy(srch.at[s], sslab)
        pltpu.sync_copy(opih.at[w], oslab)

        @pl.loop(0, 128)
        def _(i):
            ones.at[pl.ds(i, 1), pl.ds(0, 16)][...] = jnp.full(
                (1, 16), 1.0, jnp.float32)

        @pl.loop(0, 392)
        def _(i):
            halfb.at[pl.ds(i, 1), pl.ds(0, 16)][...] = jnp.full(
                (1, 16), 0.5, jnp.float32)

        @pl.loop(0, 8)
        def _(i):
            pltpu.sync_copy(halfb, accd.at[pl.ds(s * _RPT + i * 392, 392)])

        plsc.subcore_barrier()

        @pl.loop(0, _ECH // 2)
        def _(j):
            pltpu.sync_copy(ones, accd.at[sslab.at[c * (_ECH // 2) + j]],
                            add=True)

        @pl.loop(0, 13)
        def _(j):
            pltpu.sync_copy(embh.at[oslab.at[j]], obuf)
            pltpu.sync_copy(obuf, opeo.at[pl.ds(w * 1664 + j * 128, 128)])

        plsc.subcore_barrier()

        @pl.when(c == 0)
        def _():
            pltpu.sync_copy(accd.at[pl.ds(s * _RPT, _RPT)],
                            d0o.at[pl.ds(s * _RPT, _RPT)])

        @pl.when(c == 1)
        def _():
            pltpu.sync_copy(accd.at[pl.ds(s * _RPT, _RPT)],
                            d1o.at[pl.ds(s * _RPT, _RPT)])

    return k(srcp, opi, emb)


def _sc_edge(u8, y8, srcp, dstp, zrows):
    out_type = [jax.ShapeDtypeStruct((8, _NP, _HE), jnp.float32)] * 2

    @functools.partial(
        pl.kernel, mesh=_mesh(), out_type=out_type,
        compiler_params=pltpu.CompilerParams(use_tc_tiling_on_sc=False),
        scratch_types=[
            pltpu.VMEM((_ECH, 128), jnp.int32),
            pltpu.VMEM((_ECH, 128), jnp.int32),
            pltpu.VMEM((128, _HE), jnp.float32),
            pltpu.VMEM((128, _HE), jnp.float32),
            pltpu.VMEM_SHARED((_NP, _HE), jnp.float32),
            pltpu.SemaphoreType.DMA,
            pltpu.SemaphoreType.DMA,
        ])
    def k(u8h, y8h, srch, dsth, zrh, s1o, s2o,
          sslab, dslab, bufa, bufb, acc, sema, semb):
        c = lax.axis_index("c")
        s = lax.axis_index("s")
        pltpu.sync_copy(srch.at[s], sslab)
        pltpu.sync_copy(dsth.at[s], dslab)

        def sweep(tabh, gsl, ssl, outh):
            pltpu.sync_copy(zrh.at[pl.ds(s * _RPT, _RPT)],
                            acc.at[pl.ds(s * _RPT, _RPT)])
            plsc.subcore_barrier()
            pltpu.async_copy(tabh.at[gsl.at[0]], bufa, sema)

            @pl.loop(0, _ECH, step=2)
            def _(j):
                pltpu.async_copy(tabh.at[gsl.at[j + 1]], bufb, semb)
                pltpu.make_async_copy(tabh.at[gsl.at[j]], bufa, sema).wait()
                pltpu.sync_copy(bufa, acc.at[ssl.at[j]], add=True)

                @pl.when(j + 2 < _ECH)
                def _():
                    pltpu.async_copy(tabh.at[gsl.at[j + 2]], bufa, sema)

                pltpu.make_async_copy(tabh.at[gsl.at[j + 1]], bufb,
                                      semb).wait()
                pltpu.sync_copy(bufb, acc.at[ssl.at[j + 1]], add=True)

            plsc.subcore_barrier()
            pltpu.sync_copy(acc.at[pl.ds(s * _RPT, _RPT)],
                            outh.at[pl.ds(s * _RPT, _RPT)])

        def all_sweeps(qb):
            for e in range(4):
                sweep(u8h.at[qb + e], sslab, dslab, s1o.at[qb + e])
            for e in range(4):
                sweep(y8h.at[qb + e], dslab, sslab, s2o.at[qb + e])

        @pl.when(c == 0)
        def _():
            all_sweeps(0)

        @pl.when(c == 1)
        def _():
            all_sweeps(4)

    return k(u8, y8, srcp, dstp, zrows)


_WSPEC = lambda r, c: pl.BlockSpec((r, c), lambda i: (0, 0))
_BSPEC = lambda c: pl.BlockSpec((_B, c), lambda i: (i, 0))

_XYU_OUT = [jax.ShapeDtypeStruct((_NP, _H), jnp.float32)] * 3


def _tc_prenet(nf, ope, d0, d1, w1a, w1b, b1, w2, b2):
    def body(nf_r, ope_r, d0_r, d1_r, w1a_r, w1b_r, b1_r, w2_r, b2_r,
             x_o, y_o, u_o):
        h = jnp.dot(nf_r[...], w1a_r[...], preferred_element_type=jnp.float32)
        h = h + jnp.dot(ope_r[...], w1b_r[...],
                        preferred_element_type=jnp.float32) + b1_r[...]
        x = jnp.dot(_leaky(h), w2_r[...],
                    preferred_element_type=jnp.float32) + b2_r[...]
        inv = 1.0 / (d0_r[...][:, :1] + d1_r[...][:, :1])
        y = _leaky(x)
        x_o[...] = x
        y_o[...] = y
        u_o[...] = y * inv

    return pl.pallas_call(
        body, grid=(_GB,),
        in_specs=[_BSPEC(128), _BSPEC(32), _BSPEC(16), _BSPEC(16),
                  _WSPEC(128, 64), _WSPEC(32, 64), _WSPEC(1, 64),
                  _WSPEC(64, 64), _WSPEC(1, 64)],
        out_specs=[_BSPEC(64)] * 3,
        out_shape=_XYU_OUT,
    )(nf, ope, d0, d1, w1a, w1b, b1, w2, b2)


def _tc_layer(x, s1, s2, d0, d1, w1, b1, w2, b2):
    def body(x_r, s1_r, s2_r, d0_r, d1_r, w1_r, b1_r, w2_r, b2_r,
             x_o, y_o, u_o):
        inv = 1.0 / (d0_r[...][:, :1] + d1_r[...][:, :1])
        xv = x_r[...]
        y = _leaky(xv)
        u = y * inv
        z = s1_r[...] + s2_r[...] * inv + 2.0 * u + y
        hh = _leaky(jnp.dot(z, w1_r[...],
                            preferred_element_type=jnp.float32) + b1_r[...])
        h = jnp.dot(hh, w2_r[...],
                    preferred_element_type=jnp.float32) + b2_r[...]
        xn = xv + h
        yn = _leaky(xn)
        x_o[...] = xn
        y_o[...] = yn
        u_o[...] = yn * inv

    return pl.pallas_call(
        body, grid=(_GB,),
        in_specs=[_BSPEC(64)] * 3 + [_BSPEC(16)] * 2
        + [_WSPEC(64, 64), _WSPEC(1, 64), _WSPEC(64, 64), _WSPEC(1, 64)],
        out_specs=[_BSPEC(64)] * 3,
        out_shape=_XYU_OUT,
    )(x, s1, s2, d0, d1, w1, b1, w2, b2)


def _tc_final(y, pw1, pb1, pw2, pb2):
    def body(y_r, w1_r, b1_r, w2_r, b2_r, o_ref, acc):
        i = pl.program_id(0)

        @pl.when(i == 0)
        def _():
            acc[...] = jnp.zeros_like(acc)

        yv = y_r[...]
        gid = lax.broadcasted_iota(jnp.int32, (_B, 64), 0) + i * _B
        yv = jnp.where(gid < _N, yv, 0.0)
        acc[...] += jnp.sum(yv, axis=0, keepdims=True)

        @pl.when(i == _GB - 1)
        def _():
            p = acc[...]
            o = jnp.dot(_leaky(jnp.dot(p, w1_r[...],
                                       preferred_element_type=jnp.float32)
                               + b1_r[...]),
                        w2_r[...], preferred_element_type=jnp.float32)
            o_ref[...] = o + b2_r[...]

    return pl.pallas_call(
        body, grid=(_GB,),
        in_specs=[_BSPEC(64), _WSPEC(64, 64), _WSPEC(1, 64),
                  _WSPEC(64, 1), _WSPEC(1, 1)],
        out_specs=[pl.BlockSpec((1, 1), lambda i: (0, 0))],
        out_shape=[jax.ShapeDtypeStruct((1, 1), jnp.float32)],
        scratch_shapes=[pltpu.VMEM((1, 64), jnp.float32)],
    )(y, pw1, pb1, pw2, pb2)


def _to8(t):
    return t.reshape(_NP, 8, 8).transpose(1, 0, 2)


def _from8(t8):
    return t8.transpose(1, 0, 2).reshape(_NP, _H)


def kernel(op_ids, node_feats, edge_index, emb, pre_w1, pre_b1, pre_w2,
           pre_b2, gc_w1, gc_b1, gc_w2, gc_b2, post_w1, post_b1, post_w2,
           post_b2):
    n, df = node_feats.shape
    e = edge_index.shape[1]
    assert n == _N and e == _E and pre_w2.shape == (_H, _H)

    nf = jnp.pad(node_feats.astype(jnp.float32), ((0, _NP - n), (0, 0)))
    opi = jnp.pad(op_ids.astype(jnp.int32),
                  (0, _NOP - n)).reshape(32, 13, 128)
    src = edge_index[0].astype(jnp.int32)
    dst = edge_index[1].astype(jnp.int32)
    trash = n + (jnp.arange(_EP - e, dtype=jnp.int32) % (_NP - n))
    srcp = jnp.concatenate([src, trash]).reshape(16, _ECH, 128)
    dstp = jnp.concatenate([dst, trash]).reshape(16, _ECH, 128)
    zrows = jnp.zeros((_NP, _HE), jnp.float32)

    d0, d1, ope = _sc_deg_emb(srcp, opi, emb)
    x, y, u = _tc_prenet(
        nf, ope[:_NP], d0, d1, pre_w1[:df], pre_w1[df:],
        pre_b1.reshape(1, _H), pre_w2, pre_b2.reshape(1, _H))
    for i in range(gc_w1.shape[0]):
        s1_8, s2_8 = _sc_edge(_to8(u), _to8(y), srcp, dstp, zrows)
        x, y, u = _tc_layer(
            x, _from8(s1_8), _from8(s2_8), d0, d1, gc_w1[i],
            gc_b1[i].reshape(1, _H), gc_w2[i], gc_b2[i].reshape(1, _H))
    o = _tc_final(y, post_w1, post_b1.reshape(1, _H), post_w2,
                  post_b2.reshape(1, 1))
    return o[0].reshape((1,))


# speedup vs baseline: 4.0847x; 1.0481x over previous
import functools

import jax
import jax.numpy as jnp
from jax import lax
from jax.experimental import pallas as pl
from jax.experimental.pallas import tpu as pltpu
from jax.experimental.pallas import tpu_sc as plsc

_N = 50000
_E = 800000
_H = 64
_HE = 8
_NP = 50176
_NOP = 53248
_EP = 811008
_ECH = 396
_RPT = _NP // 16
_B = 3136
_GB = _NP // _B
_DH = 16768
_DT = _DH + 128
_DRT = _DT // 16
_DDR = _DH // 16


def _mesh():
    return plsc.VectorSubcoreMesh(core_axis_name="c", subcore_axis_name="s")


def _leaky(t):
    return jnp.where(t >= 0, t, 0.2 * t)


def _sc_deg_emb(srcp, opi, emb, half16):
    out_type = [
        jax.ShapeDtypeStruct((_NP, 16), jnp.float32),
        jax.ShapeDtypeStruct((_NP, 16), jnp.float32),
        jax.ShapeDtypeStruct((_NOP, 32), jnp.float32),
    ]

    @functools.partial(
        pl.kernel, mesh=_mesh(), out_type=out_type,
        compiler_params=pltpu.CompilerParams(use_tc_tiling_on_sc=False),
        scratch_types=[
            pltpu.VMEM((_ECH, 128), jnp.int32),
            pltpu.VMEM((_ECH // 2, 128), jnp.int32),
            pltpu.VMEM((13, 128), jnp.int32),
            pltpu.VMEM((128, 16), jnp.float32),
            pltpu.VMEM((128, 32), jnp.float32),
            pltpu.VMEM_SHARED((_DT, 16), jnp.float32),
        ])
    def k(srch, opih, embh, halfh, d0o, d1o, opeo, rslab, wslab, oslab,
          ones, obuf, accd):
        c = lax.axis_index("c")
        s = lax.axis_index("s")
        w = 2 * s + c
        pltpu.sync_copy(srch.at[s], rslab)
        pltpu.sync_copy(opih.at[w], oslab)

        @pl.loop(0, 128)
        def _(i):
            ones.at[pl.ds(i, 1), pl.ds(0, 16)][...] = jnp.full(
                (1, 16), 1.0, jnp.float32)

        iota16 = lax.iota(jnp.int32, 16).reshape(1, 16)
        half = _ECH // 2

        def dpass(p, base):
            @pl.loop(0, half)
            def _(r):
                for g in range(8):
                    sl = (pl.ds(c * half + r, 1), pl.ds(16 * g, 16))
                    v = rslab.at[sl][...] - base
                    ok = (v >= 0) & (v < _DH)
                    tr = (_DH + 16 * g) + iota16
                    wslab.at[pl.ds(r, 1), pl.ds(16 * g, 16)][...] = (
                        jnp.where(ok, v, tr))

            pltpu.sync_copy(halfh.at[pl.ds(s * _DRT, _DRT)],
                            accd.at[pl.ds(s * _DRT, _DRT)])
            plsc.subcore_barrier()

            @pl.loop(0, half)
            def _(j):
                pltpu.sync_copy(ones, accd.at[wslab.at[j]], add=True)

            plsc.subcore_barrier()

            @pl.when(c == 0)
            def _():
                pltpu.sync_copy(accd.at[pl.ds(s * _DDR, _DDR)],
                                d0o.at[pl.ds(base + s * _DDR, _DDR)])

            @pl.when(c == 1)
            def _():
                pltpu.sync_copy(accd.at[pl.ds(s * _DDR, _DDR)],
                                d1o.at[pl.ds(base + s * _DDR, _DDR)])

            plsc.subcore_barrier()

        dpass(0, 0)
        dpass(1, _DH)
        dpass(2, 2 * _DH)

        @pl.loop(0, 13)
        def _(j):
            pltpu.sync_copy(embh.at[oslab.at[j]], obuf)
            pltpu.sync_copy(obuf, opeo.at[pl.ds(w * 1664 + j * 128, 128)])

    return k(srcp, opi, emb, half16)


def _sc_edge(u8, y8, srcp, dstp, zrows):
    out_type = [jax.ShapeDtypeStruct((8, _NP, _HE), jnp.float32)] * 2

    @functools.partial(
        pl.kernel, mesh=_mesh(), out_type=out_type,
        compiler_params=pltpu.CompilerParams(use_tc_tiling_on_sc=False),
        scratch_types=[
            pltpu.VMEM((_ECH, 128), jnp.int32),
            pltpu.VMEM((_ECH, 128), jnp.int32),
            pltpu.VMEM((4, 128, _HE), jnp.float32),
            pltpu.VMEM_SHARED((_NP, _HE), jnp.float32),
            [pltpu.SemaphoreType.DMA] * 2,
            [pltpu.SemaphoreType.DMA] * 2,
        ])
    def k(u8h, y8h, srch, dsth, zrh, s1o, s2o,
          sslab, dslab, bufs, acc, gsem, ssem):
        c = lax.axis_index("c")
        s = lax.axis_index("s")
        pltpu.sync_copy(srch.at[s], sslab)
        pltpu.sync_copy(dsth.at[s], dslab)

        def sweep(tabh, gsl, ssl, outh):
            pltpu.sync_copy(zrh.at[pl.ds(s * _RPT, _RPT)],
                            acc.at[pl.ds(s * _RPT, _RPT)])
            plsc.subcore_barrier()
            for k in range(2):
                pltpu.async_copy(tabh.at[gsl.at[k]], bufs.at[k], gsem[k])

            @pl.loop(0, _ECH, step=4)
            def _(j):
                for k in range(4):
                    jk = j + k
                    sl = k
                    s2 = (k + 2) % 4
                    sm = k % 2
                    pltpu.make_async_copy(tabh.at[gsl.at[jk]],
                                          bufs.at[sl], gsem[sm]).wait()

                    @pl.when(jk >= 2)
                    def _():
                        pltpu.make_async_copy(
                            bufs.at[s2], acc.at[ssl.at[jk - 2]],
                            ssem[sm]).wait()

                    pltpu.async_copy(bufs.at[sl], acc.at[ssl.at[jk]],
                                     ssem[sm], add=True)

                    @pl.when(jk + 2 < _ECH)
                    def _():
                        pltpu.async_copy(tabh.at[gsl.at[jk + 2]],
                                         bufs.at[s2], gsem[sm])

            for k in range(2):
                jk = _ECH - 2 + k
                pltpu.make_async_copy(bufs.at[jk % 4],
                                      acc.at[ssl.at[jk]],
                                      ssem[jk % 2]).wait()
            plsc.subcore_barrier()
            pltpu.sync_copy(acc.at[pl.ds(s * _RPT, _RPT)],
                            outh.at[pl.ds(s * _RPT, _RPT)])

        def all_sweeps(qb):
            for e in range(4):
                sweep(u8h.at[qb + e], sslab, dslab, s1o.at[qb + e])
            for e in range(4):
                sweep(y8h.at[qb + e], dslab, sslab, s2o.at[qb + e])

        @pl.when(c == 0)
        def _():
            all_sweeps(0)

        @pl.when(c == 1)
        def _():
            all_sweeps(4)

    return k(u8, y8, srcp, dstp, zrows)


_WSPEC = lambda r, c: pl.BlockSpec((r, c), lambda i: (0, 0))
_BSPEC = lambda c: pl.BlockSpec((_B, c), lambda i: (i, 0))

_XYU_OUT = [jax.ShapeDtypeStruct((_NP, _H), jnp.float32)] * 3


def _tc_prenet(nf, ope, d0, d1, w1a, w1b, b1, w2, b2):
    def body(nf_r, ope_r, d0_r, d1_r, w1a_r, w1b_r, b1_r, w2_r, b2_r,
             x_o, y_o, u_o):
        h = jnp.dot(nf_r[...], w1a_r[...], preferred_element_type=jnp.float32)
        h = h + jnp.dot(ope_r[...], w1b_r[...],
                        preferred_element_type=jnp.float32) + b1_r[...]
        x = jnp.dot(_leaky(h), w2_r[...],
                    preferred_element_type=jnp.float32) + b2_r[...]
        inv = 1.0 / (d0_r[...][:, :1] + d1_r[...][:, :1])
        y = _leaky(x)
        x_o[...] = x
        y_o[...] = y
        u_o[...] = y * inv

    return pl.pallas_call(
        body, grid=(_GB,),
        in_specs=[_BSPEC(128), _BSPEC(32), _BSPEC(16), _BSPEC(16),
                  _WSPEC(128, 64), _WSPEC(32, 64), _WSPEC(1, 64),
                  _WSPEC(64, 64), _WSPEC(1, 64)],
        out_specs=[_BSPEC(64)] * 3,
        out_shape=_XYU_OUT,
    )(nf, ope, d0, d1, w1a, w1b, b1, w2, b2)


def _tc_layer(x, s1, s2, d0, d1, w1, b1, w2, b2):
    def body(x_r, s1_r, s2_r, d0_r, d1_r, w1_r, b1_r, w2_r, b2_r,
             x_o, y_o, u_o):
        inv = 1.0 / (d0_r[...][:, :1] + d1_r[...][:, :1])
        xv = x_r[...]
        y = _leaky(xv)
        u = y * inv
        z = s1_r[...] + s2_r[...] * inv + 2.0 * u + y
        hh = _leaky(jnp.dot(z, w1_r[...],
                            preferred_element_type=jnp.float32) + b1_r[...])
        h = jnp.dot(hh, w2_r[...],
                    preferred_element_type=jnp.float32) + b2_r[...]
        xn = xv + h
        yn = _leaky(xn)
        x_o[...] = xn
        y_o[...] = yn
        u_o[...] = yn * inv

    return pl.pallas_call(
        body, grid=(_GB,),
        in_specs=[_BSPEC(64)] * 3 + [_BSPEC(16)] * 2
        + [_WSPEC(64, 64), _WSPEC(1, 64), _WSPEC(64, 64), _WSPEC(1, 64)],
        out_specs=[_BSPEC(64)] * 3,
        out_shape=_XYU_OUT,
    )(x, s1, s2, d0, d1, w1, b1, w2, b2)


def _tc_final(y, pw1, pb1, pw2, pb2):
    def body(y_r, w1_r, b1_r, w2_r, b2_r, o_ref, acc):
        i = pl.program_id(0)

        @pl.when(i == 0)
        def _():
            acc[...] = jnp.zeros_like(acc)

        yv = y_r[...]
        gid = lax.broadcasted_iota(jnp.int32, (_B, 64), 0) + i * _B
        yv = jnp.where(gid < _N, yv, 0.0)
        acc[...] += jnp.sum(yv, axis=0, keepdims=True)

        @pl.when(i == _GB - 1)
        def _():
            p = acc[...]
            o = jnp.dot(_leaky(jnp.dot(p, w1_r[...],
                                       preferred_element_type=jnp.float32)
                               + b1_r[...]),
                        w2_r[...], preferred_element_type=jnp.float32)
            o_ref[...] = o + b2_r[...]

    return pl.pallas_call(
        body, grid=(_GB,),
        in_specs=[_BSPEC(64), _WSPEC(64, 64), _WSPEC(1, 64),
                  _WSPEC(64, 1), _WSPEC(1, 1)],
        out_specs=[pl.BlockSpec((1, 1), lambda i: (0, 0))],
        out_shape=[jax.ShapeDtypeStruct((1, 1), jnp.float32)],
        scratch_shapes=[pltpu.VMEM((1, 64), jnp.float32)],
    )(y, pw1, pb1, pw2, pb2)


def _to8(t):
    return t.reshape(_NP, 8, 8).transpose(1, 0, 2)


def _from8(t8):
    return t8.transpose(1, 0, 2).reshape(_NP, _H)


def kernel(op_ids, node_feats, edge_index, emb, pre_w1, pre_b1, pre_w2,
           pre_b2, gc_w1, gc_b1, gc_w2, gc_b2, post_w1, post_b1, post_w2,
           post_b2):
    n, df = node_feats.shape
    e = edge_index.shape[1]
    assert n == _N and e == _E and pre_w2.shape == (_H, _H)

    nf = jnp.pad(node_feats.astype(jnp.float32), ((0, _NP - n), (0, 0)))
    opi = jnp.pad(op_ids.astype(jnp.int32),
                  (0, _NOP - n)).reshape(32, 13, 128)
    src = edge_index[0].astype(jnp.int32)
    dst = edge_index[1].astype(jnp.int32)
    trash = n + (jnp.arange(_EP - e, dtype=jnp.int32) % (_NP - n))
    srcf = jnp.concatenate([src, trash])
    dstf = jnp.concatenate([dst, trash])
    srcp = srcf.reshape(16, _ECH, 128)
    dstp = dstf.reshape(16, _ECH, 128)

    zrows = jnp.zeros((_NP, _HE), jnp.float32)
    half16 = jnp.full((_NP, 16), 0.5, jnp.float32)

    d0, d1, ope = _sc_deg_emb(srcp, opi, emb, half16)
    x, y, u = _tc_prenet(
        nf, ope[:_NP], d0, d1, pre_w1[:df], pre_w1[df:],
        pre_b1.reshape(1, _H), pre_w2, pre_b2.reshape(1, _H))
    for i in range(gc_w1.shape[0]):
        s1_8, s2_8 = _sc_edge(_to8(u), _to8(y), srcp, dstp, zrows)
        x, y, u = _tc_layer(
            x, _from8(s1_8), _from8(s2_8), d0, d1, gc_w1[i],
            gc_b1[i].reshape(1, _H), gc_w2[i], gc_b2[i].reshape(1, _H))
    o = _tc_final(y, post_w1, post_b1.reshape(1, _H), post_w2,
                  post_b2.reshape(1, 1))
    return o[0].reshape((1,))
